# baseline (speedup 1.0000x reference)
import functools

import jax
import jax.numpy as jnp
from jax import lax
from jax.experimental import pallas as pl
from jax.experimental.pallas import tpu as pltpu
from jax.experimental.pallas import tpu_sc as plsc

B, L = 16384, 20
FEAT = 200
D_MODEL = 512
VPAD = 128

NC, NS = 2, 16
NW = NC * NS
TOTAL = B * L
BPW = TOTAL // NW
CHUNK = 64
NCHUNK = BPW // CHUNK
NPAIR = NCHUNK // 2


def _table_body(cbfv_ref, w_ref, b_ref, out_ref):
    acc = lax.dot_general(
        cbfv_ref[...], w_ref[...],
        dimension_numbers=(((1,), (1,)), ((), ())),
        preferred_element_type=jnp.float32,
    )
    out_ref[...] = acc + b_ref[...]


def _fuse_table(cbfv_pad, W, b2d):
    return pl.pallas_call(
        _table_body,
        out_shape=jax.ShapeDtypeStruct((VPAD, D_MODEL), jnp.float32),
    )(cbfv_pad, W, b2d)


@functools.cache
def _build_sc_gather():
    mesh = plsc.VectorSubcoreMesh(
        core_axis_name="c", subcore_axis_name="s", num_cores=NC, num_subcores=NS
    )
    return pl.kernel(
        _sc_gather_body,
        out_type=jax.ShapeDtypeStruct((TOTAL, D_MODEL), jnp.float32),
        mesh=mesh,
        scratch_types=[
            pltpu.VMEM((BPW,), jnp.int32),
            pltpu.VMEM((CHUNK, D_MODEL), jnp.float32),
            pltpu.VMEM((CHUNK, D_MODEL), jnp.float32),
            pltpu.SemaphoreType.DMA,
            pltpu.SemaphoreType.DMA,
        ],
    )


def _sc_gather_body(table_hbm, idx_hbm, out_hbm, idx_v, buf0, buf1, sem0, sem1):
    wid = lax.axis_index("s") * NC + lax.axis_index("c")
    base = wid * BPW
    pltpu.sync_copy(idx_hbm.at[pl.ds(base, BPW)], idx_v)

    def start(g, buf, sem):
        pltpu.async_copy(table_hbm.at[idx_v.at[pl.ds(g * CHUNK, CHUNK)]], buf, sem)

    def wait(buf, sem):
        pltpu.make_async_copy(table_hbm.at[pl.ds(0, CHUNK)], buf, sem).wait()

    def drain(g, buf):
        pltpu.sync_copy(buf, out_hbm.at[pl.ds(base + g * CHUNK, CHUNK)])

    start(0, buf0, sem0)

    def pair(p, carry):
        g0 = 2 * p
        start(g0 + 1, buf1, sem1)
        wait(buf0, sem0)
        drain(g0, buf0)
        start(g0 + 2, buf0, sem0)
        wait(buf1, sem1)
        drain(g0 + 1, buf1)
        return carry

    lax.fori_loop(0, NPAIR - 1, pair, 0)

    g0 = 2 * (NPAIR - 1)
    start(g0 + 1, buf1, sem1)
    wait(buf0, sem0)
    drain(g0, buf0)
    wait(buf1, sem1)
    drain(g0 + 1, buf1)


def kernel(src, cbfv, W, b):
    cbfv_pad = jnp.pad(cbfv, ((0, VPAD - cbfv.shape[0]), (0, 0)))
    table = _fuse_table(cbfv_pad, W, b.reshape(1, D_MODEL))
    idx = src.reshape(-1).astype(jnp.int32)
    out_flat = _build_sc_gather()(table, idx)
    return out_flat.reshape(B, L, D_MODEL)
